# Initial kernel scaffold; baseline (speedup 1.0000x reference)
import jax
import jax.numpy as jnp
from jax.experimental import pallas as pl


def kernel(x, edge_index, edge_attr, batch, params):
    raise NotImplementedError("write your pallas kernel here")


# speedup vs baseline: 1.0439x; 1.0439x over previous
import jax
import jax.numpy as jnp
from jax.experimental import pallas as pl

H, C, HID, NG, NL = 8, 16, 128, 256, 4


def _proj_kernel(h_ref, w_ref, o_ref):
    o_ref[...] = jnp.dot(h_ref[...], w_ref[...],
                         preferred_element_type=jnp.float32)


def _bn_relu_kernel(x_ref, g_ref, b_ref, o_ref):
    x = x_ref[...]
    mu = jnp.mean(x, axis=0, keepdims=True)
    var = jnp.mean((x - mu) * (x - mu), axis=0, keepdims=True)
    y = g_ref[...] * (x - mu) / jnp.sqrt(var + 1e-5) + b_ref[...]
    o_ref[...] = jnp.maximum(y, 0.0)


def _bn_relu_res_kernel(x_ref, g_ref, b_ref, r_ref, o_ref):
    x = x_ref[...]
    mu = jnp.mean(x, axis=0, keepdims=True)
    var = jnp.mean((x - mu) * (x - mu), axis=0, keepdims=True)
    y = g_ref[...] * (x - mu) / jnp.sqrt(var + 1e-5) + b_ref[...]
    o_ref[...] = jnp.maximum(y, 0.0) + r_ref[...]


def _gate_kernel(h_ref, w1_ref, b1_ref, w2_ref, b2_ref, o_ref):
    g1 = jnp.maximum(jnp.dot(h_ref[...], w1_ref[...],
                             preferred_element_type=jnp.float32)
                     + b1_ref[...], 0.0)
    o_ref[...] = jnp.dot(g1, w2_ref[...],
                         preferred_element_type=jnp.float32) + b2_ref[...]


def _mlp_kernel(g_ref, w1_ref, b1_ref, w2_ref, b2_ref, o_ref):
    g1 = jnp.maximum(jnp.dot(g_ref[...], w1_ref[...],
                             preferred_element_type=jnp.float32)
                     + b1_ref[...], 0.0)
    o_ref[...] = jnp.maximum(jnp.dot(g1, w2_ref[...],
                                     preferred_element_type=jnp.float32)
                             + b2_ref[...], 0.0)


def _call(body, out_shape, *args):
    return pl.pallas_call(
        body, out_shape=jax.ShapeDtypeStruct(out_shape, jnp.float32))(*args)


def _seg_softmax(logits, seg, n):
    m = jax.ops.segment_max(logits, seg, num_segments=n)
    m = jnp.where(jnp.isfinite(m), m, 0.0)
    e = jnp.exp(logits - m[seg])
    s = jax.ops.segment_sum(e, seg, num_segments=n)
    return e / (s[seg] + 1e-16)


def kernel(x, edge_index, edge_attr, batch, params):
    n = x.shape[0]
    src, dst = edge_index[0], edge_index[1]

    h = x @ params['node_W'] + params['node_b']

    Wcats, aes = [], []
    for i in range(NL):
        lp = params['layers'][i]
        W = lp['W']
        fold_src = (W.reshape(HID, H, C) * lp['att_src'][None]).sum(-1)
        fold_dst = (W.reshape(HID, H, C) * lp['att_dst'][None]).sum(-1)
        Wcat = jnp.concatenate(
            [W, fold_src, fold_dst,
             jnp.zeros((HID, 112), jnp.float32)], axis=1)
        Wcats.append(Wcat)
        fold_e = (lp['W_edge'].reshape(HID, H, C) * lp['att_edge'][None]).sum(-1)
        Me = params['edge_W'] @ fold_e
        ce = params['edge_b'] @ fold_e
        aes.append(edge_attr @ Me + ce)

    residual = h
    for i in range(NL):
        lp = params['layers'][i]
        y = _call(_proj_kernel, (n, 256), h, Wcats[i])
        xs = y[:, :HID]
        a_src = y[:, HID:HID + H]
        a_dst = y[:, HID + H:HID + 2 * H]

        alpha = jax.nn.leaky_relu(a_src[src] + a_dst[dst] + aes[i], 0.2)
        alpha = _seg_softmax(alpha, dst, n)
        msgs = xs[src].reshape(-1, H, C) * alpha[:, :, None]
        out = jax.ops.segment_sum(msgs, dst, num_segments=n)
        h_new = out.reshape(n, HID) + lp['bias']

        g2 = lp['bn_g'].reshape(1, HID)
        b2 = lp['bn_b'].reshape(1, HID)
        if i > 0 and i % 2 == 0:
            h_new = _call(_bn_relu_res_kernel, (n, HID), h_new, g2, b2,
                          residual)
            residual = h_new
        else:
            h_new = _call(_bn_relu_kernel, (n, HID), h_new, g2, b2)
        h = h_new

    ones = jnp.ones((n,), dtype=x.dtype)
    cnt = jax.ops.segment_sum(ones, batch, num_segments=NG)
    x_mean = jax.ops.segment_sum(h, batch, num_segments=NG) \
        / jnp.maximum(cnt, 1.0)[:, None]
    x_max = jax.ops.segment_max(h, batch, num_segments=NG)
    x_max = jnp.where(jnp.isfinite(x_max), x_max, 0.0)

    w2p = jnp.zeros((HID // 2, 128), jnp.float32).at[:, 0].set(
        params['gate_W2'][:, 0])
    b2p = jnp.zeros((1, 128), jnp.float32).at[0, 0].set(params['gate_b2'][0])
    gate = _call(_gate_kernel, (n, 128), h,
                 params['gate_W1'], params['gate_b1'].reshape(1, -1),
                 w2p, b2p)[:, 0]
    gw = _seg_softmax(gate, batch, NG)
    x_att = jax.ops.segment_sum(gw[:, None] * h, batch, num_segments=NG)

    g = jnp.concatenate([x_mean, x_max, x_att], axis=1)
    g = _call(_mlp_kernel, (NG, HID), g,
              params['mlp_W1'], params['mlp_b1'].reshape(1, -1),
              params['mlp_W2'], params['mlp_b2'].reshape(1, -1))
    return g


# speedup vs baseline: 12.3383x; 11.8190x over previous
import jax
import jax.numpy as jnp
from jax.experimental import pallas as pl

H, C, HID, NG, NL = 8, 16, 128, 256, 4


def _proj_kernel(h_ref, w_ref, o_ref):
    o_ref[...] = jnp.dot(h_ref[...], w_ref[...],
                         preferred_element_type=jnp.float32)


def _bn_relu_kernel(x_ref, g_ref, b_ref, o_ref):
    x = x_ref[...]
    mu = jnp.mean(x, axis=0, keepdims=True)
    var = jnp.mean((x - mu) * (x - mu), axis=0, keepdims=True)
    y = g_ref[...] * (x - mu) / jnp.sqrt(var + 1e-5) + b_ref[...]
    o_ref[...] = jnp.maximum(y, 0.0)


def _bn_relu_res_kernel(x_ref, g_ref, b_ref, r_ref, o_ref):
    x = x_ref[...]
    mu = jnp.mean(x, axis=0, keepdims=True)
    var = jnp.mean((x - mu) * (x - mu), axis=0, keepdims=True)
    y = g_ref[...] * (x - mu) / jnp.sqrt(var + 1e-5) + b_ref[...]
    o_ref[...] = jnp.maximum(y, 0.0) + r_ref[...]


def _gate_kernel(h_ref, w1_ref, b1_ref, w2_ref, b2_ref, o_ref):
    g1 = jnp.maximum(jnp.dot(h_ref[...], w1_ref[...],
                             preferred_element_type=jnp.float32)
                     + b1_ref[...], 0.0)
    o_ref[...] = jnp.dot(g1, w2_ref[...],
                         preferred_element_type=jnp.float32) + b2_ref[...]


def _mlp_kernel(g_ref, w1_ref, b1_ref, w2_ref, b2_ref, o_ref):
    g1 = jnp.maximum(jnp.dot(g_ref[...], w1_ref[...],
                             preferred_element_type=jnp.float32)
                     + b1_ref[...], 0.0)
    o_ref[...] = jnp.maximum(jnp.dot(g1, w2_ref[...],
                                     preferred_element_type=jnp.float32)
                             + b2_ref[...], 0.0)


def _call(body, out_shape, *args):
    return pl.pallas_call(
        body, out_shape=jax.ShapeDtypeStruct(out_shape, jnp.float32))(*args)


def kernel(x, edge_index, edge_attr, batch, params):
    n = x.shape[0]
    src, dst = edge_index[0], edge_index[1]

    h = x @ params['node_W'] + params['node_b']

    Wcats, aes = [], []
    for i in range(NL):
        lp = params['layers'][i]
        W = lp['W']
        fold_src = (W.reshape(HID, H, C) * lp['att_src'][None]).sum(-1)
        fold_dst = (W.reshape(HID, H, C) * lp['att_dst'][None]).sum(-1)
        Wcat = jnp.concatenate(
            [W, fold_src, fold_dst,
             jnp.zeros((HID, 112), jnp.float32)], axis=1)
        Wcats.append(Wcat)
        fold_e = (lp['W_edge'].reshape(HID, H, C) * lp['att_edge'][None]).sum(-1)
        Me = params['edge_W'] @ fold_e
        ce = params['edge_b'] @ fold_e
        aes.append(edge_attr @ Me + ce)

    residual = h
    for i in range(NL):
        lp = params['layers'][i]
        y = _call(_proj_kernel, (n, 256), h, Wcats[i])

        yg = y[src]
        a_dstE = y[:, HID + H:HID + 2 * H][dst]
        logits = jax.nn.leaky_relu(
            yg[:, HID:HID + H] + a_dstE + aes[i], 0.2)
        e = jnp.exp(logits)
        msgs = (yg[:, :HID].reshape(-1, H, C)
                * e[:, :, None]).reshape(-1, HID)
        sc = jax.ops.segment_sum(
            jnp.concatenate([e, msgs], axis=1), dst, num_segments=n)
        out = (sc[:, H:].reshape(n, H, C)
               / (sc[:, :H, None] + 1e-16)).reshape(n, HID)
        h_new = out + lp['bias']

        g2 = lp['bn_g'].reshape(1, HID)
        b2 = lp['bn_b'].reshape(1, HID)
        if i > 0 and i % 2 == 0:
            h_new = _call(_bn_relu_res_kernel, (n, HID), h_new, g2, b2,
                          residual)
            residual = h_new
        else:
            h_new = _call(_bn_relu_kernel, (n, HID), h_new, g2, b2)
        h = h_new

    w2p = jnp.zeros((HID // 2, 128), jnp.float32).at[:, 0].set(
        params['gate_W2'][:, 0])
    b2p = jnp.zeros((1, 128), jnp.float32).at[0, 0].set(params['gate_b2'][0])
    gate = _call(_gate_kernel, (n, 128), h,
                 params['gate_W1'], params['gate_b1'].reshape(1, -1),
                 w2p, b2p)[:, 0]
    eg = jnp.exp(gate)[:, None]
    ones = jnp.ones((n, 1), dtype=x.dtype)
    pooled = jax.ops.segment_sum(
        jnp.concatenate([ones, h, eg, eg * h], axis=1),
        batch, num_segments=NG)
    cnt = pooled[:, 0]
    x_mean = pooled[:, 1:1 + HID] / jnp.maximum(cnt, 1.0)[:, None]
    x_max = jax.ops.segment_max(h, batch, num_segments=NG)
    x_max = jnp.where(jnp.isfinite(x_max), x_max, 0.0)
    x_att = pooled[:, 2 + HID:] / (pooled[:, 1 + HID:2 + HID] + 1e-16)

    g = jnp.concatenate([x_mean, x_max, x_att], axis=1)
    g = _call(_mlp_kernel, (NG, HID), g,
              params['mlp_W1'], params['mlp_b1'].reshape(1, -1),
              params['mlp_W2'], params['mlp_b2'].reshape(1, -1))
    return g
